# baseline (speedup 1.0000x reference)
import jax
import jax.numpy as jnp
from jax import lax
from jax.experimental import pallas as pl
from jax.experimental.pallas import tpu as pltpu
from jax.experimental.pallas import tpu_sc as plsc

BATCH = 16384
FIELDS = 100
STRIDE = 101
NC = 2
NS = 16
NW = NC * NS
ROWS_W = BATCH // NW
LANES = 16
KROWS = 64
NCHUNK = ROWS_W // KROWS
CPACK = KROWS * STRIDE


def _body(x_hbm, tab_hbm, bias_hbm, out_hbm, stg_v, idx_v, vals_v, out_v,
          bias_v, sem_s, sem_g):
    wid = lax.axis_index("s") * NC + lax.axis_index("c")
    row_base = wid * ROWS_W

    pltpu.sync_copy(bias_hbm, bias_v)
    bias_vec = bias_v[...]
    lane = lax.iota(jnp.int32, LANES)
    tail_ld = lane < 4
    tail_st = lane < 5

    def parity(c):
        return lax.rem(c, 2)

    def stage(c):
        b = parity(c)
        pltpu.async_copy(x_hbm.at[pl.ds(row_base + c * KROWS, KROWS)],
                         stg_v.at[pl.ds(b * KROWS, KROWS)], sem_s.at[b])

    def stage_wait(c):
        b = parity(c)
        pltpu.make_async_copy(x_hbm.at[pl.ds(row_base, KROWS)],
                              stg_v.at[pl.ds(b * KROWS, KROWS)],
                              sem_s.at[b]).wait()

    def repack(c):
        b = parity(c)

        def row(r, carry):
            src_row = jnp.zeros((LANES,), jnp.int32) + (b * KROWS + r)
            dst0 = b * CPACK + r * STRIDE
            for j0 in range(0, 96, 16):
                v = plsc.load_gather(stg_v, [src_row, lane + j0])
                plsc.store_scatter(idx_v, [lane + (dst0 + j0)], v)
            vt = plsc.load_gather(stg_v, [src_row, lane + 96], mask=tail_ld)
            vt = jnp.where(tail_ld, vt, 0)
            plsc.store_scatter(idx_v, [lane + (dst0 + 96)], vt, mask=tail_st)
            return carry

        lax.fori_loop(0, KROWS, row, 0)

    def fire_gather(c):
        b = parity(c)
        pltpu.async_copy(tab_hbm.at[idx_v.at[pl.ds(b * CPACK, CPACK)]],
                         vals_v.at[pl.ds(b * CPACK, CPACK)], sem_g.at[b])

    def gather_wait(c):
        b = parity(c)
        pltpu.make_async_copy(tab_hbm.at[idx_v.at[pl.ds(b * CPACK, CPACK)]],
                              vals_v.at[pl.ds(b * CPACK, CPACK)],
                              sem_g.at[b]).wait()

    def reduce(c):
        b = parity(c)

        def grp(i, carry):
            rows = (lane + i * LANES) * STRIDE + b * CPACK

            def inner(j, acc):
                return acc + plsc.load_gather(vals_v, [rows + j])

            acc = lax.fori_loop(0, FIELDS, inner,
                                jnp.zeros((LANES,), jnp.float32), unroll=4)
            z = acc + bias_vec
            out_v[pl.ds(c * KROWS + i * LANES, LANES)] = \
                1.0 / (1.0 + jnp.exp(-z))
            return carry

        lax.fori_loop(0, KROWS // LANES, grp, 0)

    stage(0)
    stage(1)
    stage_wait(0)
    repack(0)
    fire_gather(0)

    def step(c, carry):
        @pl.when(c + 1 < NCHUNK)
        def _():
            stage_wait(c + 1)
            repack(c + 1)
            fire_gather(c + 1)

        @pl.when(c + 2 < NCHUNK)
        def _():
            stage(c + 2)

        gather_wait(c)
        reduce(c)
        return carry

    lax.fori_loop(0, NCHUNK, step, 0)
    pltpu.sync_copy(out_v, out_hbm.at[pl.ds(row_base, ROWS_W)])


@jax.jit
def _run(x2d, table_flat, bias16):
    mesh = plsc.VectorSubcoreMesh(core_axis_name="c", subcore_axis_name="s")
    f = pl.kernel(
        _body,
        out_type=jax.ShapeDtypeStruct((BATCH,), jnp.float32),
        mesh=mesh,
        scratch_types=[
            pltpu.VMEM((2 * KROWS, FIELDS), jnp.int32),
            pltpu.VMEM((2 * CPACK,), jnp.int32),
            pltpu.VMEM((2 * CPACK,), jnp.float32),
            pltpu.VMEM((ROWS_W,), jnp.float32),
            pltpu.VMEM((LANES,), jnp.float32),
            pltpu.SemaphoreType.DMA((2,)),
            pltpu.SemaphoreType.DMA((2,)),
        ],
        compiler_params=pltpu.CompilerParams(needs_layout_passes=False),
    )
    return f(x2d, table_flat, bias16)


def kernel(x, table, bias):
    return _run(x, table.reshape(-1), jnp.broadcast_to(bias, (LANES,)))


# speedup vs baseline: 1.2562x; 1.2562x over previous
import jax
import jax.numpy as jnp
from jax import lax
from jax.experimental import pallas as pl
from jax.experimental.pallas import tpu as pltpu
from jax.experimental.pallas import tpu_sc as plsc

BATCH = 16384
FIELDS = 100
NC = 2
NS = 16
NW = NC * NS
ROWS_W = BATCH // NW
CHUNK = ROWS_W * FIELDS
HROWS = ROWS_W // 2
HALF = CHUNK // 2
LANES = 16


def _body(x_hbm, tab_hbm, bias_hbm, out_hbm, idx_v, vals_v, out_v, bias_v,
          sems):
    wid = lax.axis_index("s") * NC + lax.axis_index("c")
    base = wid * CHUNK

    pltpu.sync_copy(bias_hbm, bias_v)
    pltpu.sync_copy(x_hbm.at[pl.ds(base, CHUNK)], idx_v)

    def fire(h):
        pltpu.async_copy(tab_hbm.at[idx_v.at[pl.ds(h * HALF, HALF)]],
                         vals_v.at[pl.ds(h * HALF, HALF)], sems.at[h])

    def wait(h):
        pltpu.make_async_copy(tab_hbm.at[idx_v.at[pl.ds(h * HALF, HALF)]],
                              vals_v.at[pl.ds(h * HALF, HALF)],
                              sems.at[h]).wait()

    fire(0)
    fire(1)

    bias_vec = bias_v[...]
    lane_off = lax.iota(jnp.int32, LANES) * FIELDS

    def reduce_half(h):
        def outer(i, carry):
            row0 = h * HROWS + i * LANES
            ibase = lane_off + row0 * FIELDS

            def inner(j, acc):
                return acc + plsc.load_gather(vals_v, [ibase + j])

            acc = lax.fori_loop(0, FIELDS, inner,
                                jnp.zeros((LANES,), jnp.float32), unroll=4)
            z = acc + bias_vec
            out_v[pl.ds(row0, LANES)] = 1.0 / (1.0 + jnp.exp(-z))
            return carry

        lax.fori_loop(0, HROWS // LANES, outer, 0)

    wait(0)
    reduce_half(0)
    wait(1)
    reduce_half(1)
    pltpu.sync_copy(out_v, out_hbm.at[pl.ds(wid * ROWS_W, ROWS_W)])


@jax.jit
def _run(x_flat, table_flat, bias16):
    mesh = plsc.VectorSubcoreMesh(core_axis_name="c", subcore_axis_name="s")
    f = pl.kernel(
        _body,
        out_type=jax.ShapeDtypeStruct((BATCH,), jnp.float32),
        mesh=mesh,
        scratch_types=[
            pltpu.VMEM((CHUNK,), jnp.int32),
            pltpu.VMEM((CHUNK,), jnp.float32),
            pltpu.VMEM((ROWS_W,), jnp.float32),
            pltpu.VMEM((LANES,), jnp.float32),
            pltpu.SemaphoreType.DMA((2,)),
        ],
        compiler_params=pltpu.CompilerParams(needs_layout_passes=False),
    )
    return f(x_flat, table_flat, bias16)


def kernel(x, table, bias):
    return _run(x.reshape(-1), table.reshape(-1),
                jnp.broadcast_to(bias, (LANES,)))


# speedup vs baseline: 1.7498x; 1.3929x over previous
import jax
import jax.numpy as jnp
from jax import lax
from jax.experimental import pallas as pl
from jax.experimental.pallas import tpu as pltpu
from jax.experimental.pallas import tpu_sc as plsc

BATCH = 16384
FIELDS = 100
VOCAB = 1000000
NC = 2
NS = 16
NW = NC * NS
ROWS_W = BATCH // NW
CHUNK = ROWS_W * FIELDS
QROWS = 128
Q = QROWS * FIELDS
NQ = ROWS_W // QROWS
SSH = 64000
HOP = Q
LANES = 16


def _body(x_hbm, tab_hbm, bias_hbm, out_hbm, shared_v, idx_v, vals_v, out_v,
          bias_v, sems_i, sems_g, sem_t):
    wid = lax.axis_index("s") * NC + lax.axis_index("c")
    sid = lax.axis_index("s")
    base = wid * CHUNK

    def stage_idx(q):
        b = q % 2
        pltpu.async_copy(x_hbm.at[pl.ds(base + q * Q, Q)],
                         idx_v.at[pl.ds(b * Q, Q)], sems_i.at[b])

    def wait_idx(q):
        b = q % 2
        pltpu.make_async_copy(x_hbm.at[pl.ds(base + q * Q, Q)],
                              idx_v.at[pl.ds(b * Q, Q)], sems_i.at[b]).wait()

    stage_idx(0)
    stage_idx(1)
    pltpu.sync_copy(bias_hbm, bias_v)

    off = jnp.minimum(sid * SSH, VOCAB - SSH)
    for h in range(SSH // HOP):
        o2 = off + h * HOP
        b = h % 2
        pltpu.async_copy(tab_hbm.at[pl.ds(o2, HOP)],
                         vals_v.at[pl.ds(b * Q, Q)], sem_t).wait()
        pltpu.async_copy(vals_v.at[pl.ds(b * Q, Q)],
                         shared_v.at[pl.ds(o2, HOP)], sem_t).wait()
    plsc.subcore_barrier()

    def fire_gather(q):
        b = q % 2
        pltpu.async_copy(shared_v.at[idx_v.at[pl.ds(b * Q, Q)]],
                         vals_v.at[pl.ds(b * Q, Q)], sems_g.at[b])

    def wait_gather(q):
        b = q % 2
        pltpu.make_async_copy(shared_v.at[idx_v.at[pl.ds(b * Q, Q)]],
                              vals_v.at[pl.ds(b * Q, Q)], sems_g.at[b]).wait()

    bias_vec = bias_v[...]
    lane_off = lax.iota(jnp.int32, LANES) * FIELDS

    def reduce_quarter(q):
        b = q % 2

        def outer(i, carry):
            ibase = lane_off + (b * Q + i * LANES * FIELDS)

            def inner(j, acc):
                return acc + plsc.load_gather(vals_v, [ibase + j])

            acc = lax.fori_loop(0, FIELDS, inner,
                                jnp.zeros((LANES,), jnp.float32), unroll=4)
            z = acc + bias_vec
            out_v[pl.ds(q * QROWS + i * LANES, LANES)] = \
                1.0 / (1.0 + jnp.exp(-z))
            return carry

        lax.fori_loop(0, QROWS // LANES, outer, 0)

    wait_idx(0)
    fire_gather(0)
    wait_idx(1)
    fire_gather(1)
    for q in range(NQ):
        wait_gather(q)
        if q + 2 < NQ:
            stage_idx(q + 2)
        reduce_quarter(q)
        if q + 2 < NQ:
            wait_idx(q + 2)
            fire_gather(q + 2)
    pltpu.sync_copy(out_v, out_hbm.at[pl.ds(wid * ROWS_W, ROWS_W)])


@jax.jit
def _run(x_flat, table_flat, bias16):
    mesh = plsc.VectorSubcoreMesh(core_axis_name="c", subcore_axis_name="s")
    f = pl.kernel(
        _body,
        out_type=jax.ShapeDtypeStruct((BATCH,), jnp.float32),
        mesh=mesh,
        scratch_types=[
            pltpu.VMEM_SHARED((VOCAB,), jnp.float32),
            pltpu.VMEM((2 * Q,), jnp.int32),
            pltpu.VMEM((2 * Q,), jnp.float32),
            pltpu.VMEM((ROWS_W,), jnp.float32),
            pltpu.VMEM((LANES,), jnp.float32),
            pltpu.SemaphoreType.DMA((2,)),
            pltpu.SemaphoreType.DMA((2,)),
            pltpu.SemaphoreType.DMA,
        ],
        compiler_params=pltpu.CompilerParams(needs_layout_passes=False),
    )
    return f(x_flat, table_flat, bias16)


def kernel(x, table, bias):
    return _run(x.reshape(-1), table.reshape(-1),
                jnp.broadcast_to(bias, (LANES,)))
